# baseline (speedup 1.0000x reference)
import functools

import jax
import jax.numpy as jnp
from jax import lax
from jax.experimental import pallas as pl
from jax.experimental.pallas import tpu as pltpu
from jax.experimental.pallas import tpu_sc as plsc

_NC = 2
_NS = 16
_NW = _NC * _NS
_L = 16
_CHUNK = 128


def _cbow_body(n_per_w, D, idx0_hbm, idx1_hbm, idx2_hbm, emb_in_hbm,
               emb_w_hbm, emb_b_hbm, out_hbm,
               idx0_v, idx1_v, idx2_v, e0_v, e1_v, w_v, b_v, out_v, sem):
    wid = lax.axis_index("s") * _NC + lax.axis_index("c")
    base = wid * n_per_w

    pltpu.sync_copy(idx0_hbm.at[pl.ds(base, n_per_w)], idx0_v)
    pltpu.sync_copy(idx1_hbm.at[pl.ds(base, n_per_w)], idx1_v)
    pltpu.sync_copy(idx2_hbm.at[pl.ds(base, n_per_w)], idx2_v)

    copies = []
    for k in range(0, n_per_w, _CHUNK):
        sl = pl.ds(k, _CHUNK)
        copies.append(pltpu.async_copy(
            emb_in_hbm.at[idx0_v.at[sl]], e0_v.at[sl], sem))
        copies.append(pltpu.async_copy(
            emb_in_hbm.at[idx1_v.at[sl]], e1_v.at[sl], sem))
        copies.append(pltpu.async_copy(
            emb_w_hbm.at[idx2_v.at[sl]], w_v.at[sl], sem))
        copies.append(pltpu.async_copy(
            emb_b_hbm.at[idx2_v.at[sl]], b_v.at[sl], sem))
    for c in copies:
        c.wait()

    lanes = lax.iota(jnp.int32, _L)

    def group(g, _):
        rows = g * _L + lanes
        acc = jnp.zeros((_L,), jnp.float32)
        for d in range(D):
            col = jnp.full((_L,), d, jnp.int32)
            a0 = plsc.load_gather(e0_v, [rows, col])
            a1 = plsc.load_gather(e1_v, [rows, col])
            aw = plsc.load_gather(w_v, [rows, col])
            acc = acc + (a0 + a1) * aw
        logit = acc * 0.5 + b_v[pl.ds(g * _L, _L)]
        out_v[pl.ds(g * _L, _L)] = 1.0 / (1.0 + jnp.exp(-logit))
        return _

    lax.fori_loop(0, n_per_w // _L, group, None)

    pltpu.sync_copy(out_v, out_hbm.at[pl.ds(base, n_per_w)])


def kernel(x, emb_in, emb_out_w, emb_out_b):
    B = x.shape[0]
    V, D = emb_in.shape
    n_per_w = B // _NW

    idx0 = x[:, 0]
    idx1 = x[:, 1]
    idx2 = x[:, 2]
    b_flat = emb_out_b.reshape(V)

    mesh = plsc.VectorSubcoreMesh(core_axis_name="c", subcore_axis_name="s")
    run = pl.kernel(
        functools.partial(_cbow_body, n_per_w, D),
        out_type=jax.ShapeDtypeStruct((B,), jnp.float32),
        mesh=mesh,
        scratch_types=[
            pltpu.VMEM((n_per_w,), jnp.int32),
            pltpu.VMEM((n_per_w,), jnp.int32),
            pltpu.VMEM((n_per_w,), jnp.int32),
            pltpu.VMEM((n_per_w, D), jnp.float32),
            pltpu.VMEM((n_per_w, D), jnp.float32),
            pltpu.VMEM((n_per_w, D), jnp.float32),
            pltpu.VMEM((n_per_w,), jnp.float32),
            pltpu.VMEM((n_per_w,), jnp.float32),
            pltpu.SemaphoreType.DMA,
        ],
        compiler_params=pltpu.CompilerParams(
            needs_layout_passes=False, use_tc_tiling_on_sc=False),
    )
    out = run(idx0, idx1, idx2, emb_in, emb_out_w, b_flat)
    return out.reshape(B, 1)
